# baseline (speedup 1.0000x reference)
import functools

import jax
import jax.numpy as jnp
from jax import lax
from jax.experimental import pallas as pl
from jax.experimental.pallas import tpu as pltpu
from jax.experimental.pallas import tpu_sc as plsc

VOCAB_ = 1000
EMB_ = 64
NW = 32
CHUNK = 640
NBUF = 3

QN = 1024 * 20 * 20
CN = 1024 * 50 * 20


def _gather_loop(idx_hbm, out_hbm, table_hbm, idx_v, rows_v, gsem, ssem,
                 base, nchunks):
    def step(g, carry):
        b = g % NBUF
        @pl.when(g >= NBUF)
        def _():
            pltpu.make_async_copy(
                rows_v.at[b], out_hbm.at[pl.ds(base, CHUNK)], ssem.at[b]
            ).wait()

        pltpu.sync_copy(idx_hbm.at[pl.ds(base + g * CHUNK, CHUNK)],
                        idx_v.at[b])
        pltpu.async_copy(table_hbm.at[idx_v.at[b]], rows_v.at[b], gsem.at[b])

        @pl.when(g >= 1)
        def _():
            pb = (g - 1) % NBUF
            pltpu.make_async_copy(
                table_hbm.at[idx_v.at[pb]], rows_v.at[pb], gsem.at[pb]
            ).wait()
            pltpu.async_copy(
                rows_v.at[pb],
                out_hbm.at[pl.ds(base + (g - 1) * CHUNK, CHUNK)],
                ssem.at[pb])

        return carry

    lax.fori_loop(0, nchunks, step, 0)
    lb = (nchunks - 1) % NBUF
    pltpu.make_async_copy(
        table_hbm.at[idx_v.at[lb]], rows_v.at[lb], gsem.at[lb]).wait()
    pltpu.async_copy(
        rows_v.at[lb],
        out_hbm.at[pl.ds(base + (nchunks - 1) * CHUNK, CHUNK)],
        ssem.at[lb])
    for k in range(NBUF):
        b = (nchunks - 1 - k) % NBUF
        pltpu.make_async_copy(
            rows_v.at[b], out_hbm.at[pl.ds(base, CHUNK)], ssem.at[b]
        ).wait()


def _body(q_hbm, c_hbm, table_hbm, qout_hbm, cout_hbm,
          idx_v, rows_v, gsem, ssem):
    wid = lax.axis_index("s") * 2 + lax.axis_index("c")
    q_per_w = QN // NW
    c_per_w = CN // NW
    _gather_loop(q_hbm, qout_hbm, table_hbm, idx_v, rows_v, gsem, ssem,
                 wid * q_per_w, q_per_w // CHUNK)
    _gather_loop(c_hbm, cout_hbm, table_hbm, idx_v, rows_v, gsem, ssem,
                 wid * c_per_w, c_per_w // CHUNK)


@jax.jit
def _run(qidx, cidx, table):
    mesh = plsc.VectorSubcoreMesh(core_axis_name="c", subcore_axis_name="s")
    f = pl.kernel(
        _body,
        out_type=(
            jax.ShapeDtypeStruct((QN, EMB_), jnp.float32),
            jax.ShapeDtypeStruct((CN, EMB_), jnp.float32),
        ),
        mesh=mesh,
        scratch_types=[
            pltpu.VMEM((NBUF, CHUNK), jnp.int32),
            pltpu.VMEM((NBUF, CHUNK, EMB_), jnp.float32),
            pltpu.SemaphoreType.DMA((NBUF,)),
            pltpu.SemaphoreType.DMA((NBUF,)),
        ],
        compiler_params=pltpu.CompilerParams(use_tc_tiling_on_sc=False),
    )
    return f(qidx, cidx, table)


def kernel(question, context, char_table):
    qshape = question.shape + (EMB_,)
    cshape = context.shape + (EMB_,)
    qidx = question.reshape(-1).astype(jnp.int32)
    cidx = context.reshape(-1).astype(jnp.int32)
    q_emb, c_emb = _run(qidx, cidx, char_table)
    return (q_emb.reshape(qshape), c_emb.reshape(cshape))


# speedup vs baseline: 1.7979x; 1.7979x over previous
import jax
import jax.numpy as jnp
from jax import lax
from jax.experimental import pallas as pl
from jax.experimental.pallas import tpu as pltpu
from jax.experimental.pallas import tpu_sc as plsc

VOCAB_ = 1000
EMB_ = 64
NW = 32
B_ = 1024
QR = 20 * 20
CR = 50 * 20
NBLK = B_ // 128
TV = EMB_ * B_


def _run_rows(idx_hbm, out_hbm, tv, idxv, stg, isem, ssem, lo, hi, cnt0,
              drain_dst):
    def row(r, cnt):
        h = pl.multiple_of((r % 2) * B_, B_)
        @pl.when(r + 1 < hi)
        def _():
            nh = ((r + 1) % 2) * B_
            pltpu.async_copy(
                idx_hbm.at[pl.ds(pl.multiple_of((r + 1) * B_, B_), B_)],
                idxv.at[pl.ds(pl.multiple_of(nh, B_), B_)],
                isem.at[(r + 1) % 2])

        pltpu.make_async_copy(
            idx_hbm.at[pl.ds(pl.multiple_of(r * B_, B_), B_)],
            idxv.at[pl.ds(h, B_)], isem.at[r % 2]).wait()

        def blk(k, cnt):
            s = cnt % 2
            b0 = pl.multiple_of(k * 128, 128)
            iv = [idxv[pl.ds(h + k * 128 + 16 * j, 16)] for j in range(8)]

            @pl.when(cnt >= 2)
            def _():
                pltpu.make_async_copy(stg.at[s], drain_dst, ssem.at[s]).wait()

            def erow(e, c):
                rowref = tv.at[pl.ds(pl.multiple_of(e * B_, B_), B_)]
                for j in range(8):
                    stg[s, e, pl.ds(16 * j, 16)] = plsc.load_gather(
                        rowref, [iv[j]])
                return c

            lax.fori_loop(0, EMB_, erow, 0, unroll=4)
            pltpu.async_copy(stg.at[s], out_hbm.at[r, :, pl.ds(b0, 128)],
                             ssem.at[s])
            return cnt + 1

        return lax.fori_loop(0, NBLK, blk, cnt)

    pltpu.async_copy(
        idx_hbm.at[pl.ds(pl.multiple_of(lo * B_, B_), B_)],
        idxv.at[pl.ds(pl.multiple_of((lo % 2) * B_, B_), B_)],
        isem.at[lo % 2])
    return lax.fori_loop(lo, hi, row, cnt0)


def _body(qT_hbm, cT_hbm, tT_hbm, qout_hbm, cout_hbm,
          tv, idxv, stg, isem, ssem):
    wid = lax.axis_index("s") * 2 + lax.axis_index("c")

    pltpu.sync_copy(tT_hbm, tv)

    drain = qout_hbm.at[0, :, pl.ds(0, 128)]
    cnt = _run_rows(qT_hbm, qout_hbm, tv, idxv, stg, isem, ssem,
                    (wid * QR) // NW, ((wid + 1) * QR) // NW, 0, drain)
    cnt = _run_rows(cT_hbm, cout_hbm, tv, idxv, stg, isem, ssem,
                    (wid * CR) // NW, ((wid + 1) * CR) // NW, cnt, drain)

    for s in range(2):
        pltpu.make_async_copy(stg.at[s], drain, ssem.at[s]).wait()


@jax.jit
def _run(qT, cT, tT):
    mesh = plsc.VectorSubcoreMesh(core_axis_name="c", subcore_axis_name="s")
    f = pl.kernel(
        _body,
        out_type=(
            jax.ShapeDtypeStruct((QR, EMB_, B_), jnp.float32),
            jax.ShapeDtypeStruct((CR, EMB_, B_), jnp.float32),
        ),
        mesh=mesh,
        scratch_types=[
            pltpu.VMEM((TV,), jnp.float32),
            pltpu.VMEM((2 * B_,), jnp.int32),
            pltpu.VMEM((2, EMB_, 128), jnp.float32),
            pltpu.SemaphoreType.DMA((2,)),
            pltpu.SemaphoreType.DMA((2,)),
        ],
        compiler_params=pltpu.CompilerParams(use_tc_tiling_on_sc=True,
                                             needs_layout_passes=False),
    )
    return f(qT, cT, tT)


def kernel(question, context, char_table):
    qT = jnp.transpose(question, (1, 2, 0)).reshape(-1).astype(jnp.int32)
    cT = jnp.transpose(context, (1, 2, 0)).reshape(-1).astype(jnp.int32)
    tT = jnp.pad(char_table.T, ((0, 0), (0, B_ - VOCAB_))).reshape(-1)
    qoT, coT = _run(qT, cT, tT)
    q_emb = qoT.reshape(20, 20, EMB_, B_).transpose(3, 0, 1, 2)
    c_emb = coT.reshape(50, 20, EMB_, B_).transpose(3, 0, 1, 2)
    return (q_emb, c_emb)


# speedup vs baseline: 1.8005x; 1.0014x over previous
import jax
import jax.numpy as jnp
from jax import lax
from jax.experimental import pallas as pl
from jax.experimental.pallas import tpu as pltpu
from jax.experimental.pallas import tpu_sc as plsc

VOCAB_ = 1000
EMB_ = 64
NW = 32
B_ = 1024
QR = 20 * 20
CR = 50 * 20
NBLK = B_ // 128
NS = 6
TV = EMB_ * B_


def _run_rows(idx_hbm, out_hbm, tv, idxv, stg, isem, ssem, lo, hi, cnt0,
              drain_dst):
    def row(r, cnt):
        h = pl.multiple_of((r % 2) * B_, B_)
        @pl.when(r + 1 < hi)
        def _():
            nh = ((r + 1) % 2) * B_
            pltpu.async_copy(
                idx_hbm.at[pl.ds(pl.multiple_of((r + 1) * B_, B_), B_)],
                idxv.at[pl.ds(pl.multiple_of(nh, B_), B_)],
                isem.at[(r + 1) % 2])

        pltpu.make_async_copy(
            idx_hbm.at[pl.ds(pl.multiple_of(r * B_, B_), B_)],
            idxv.at[pl.ds(h, B_)], isem.at[r % 2]).wait()

        def blk(k, cnt):
            s = cnt % NS
            b0 = pl.multiple_of(k * 128, 128)
            iv = [idxv[pl.ds(h + k * 128 + 16 * j, 16)] for j in range(8)]

            @pl.when(cnt >= NS)
            def _():
                pltpu.make_async_copy(stg.at[s], drain_dst, ssem.at[s]).wait()

            def erow(e, c):
                rowref = tv.at[pl.ds(pl.multiple_of(e * B_, B_), B_)]
                for j in range(8):
                    stg[s, e, pl.ds(16 * j, 16)] = plsc.load_gather(
                        rowref, [iv[j]])
                return c

            lax.fori_loop(0, EMB_, erow, 0, unroll=4)
            pltpu.async_copy(stg.at[s], out_hbm.at[r, :, pl.ds(b0, 128)],
                             ssem.at[s])
            return cnt + 1

        return lax.fori_loop(0, NBLK, blk, cnt)

    pltpu.async_copy(
        idx_hbm.at[pl.ds(pl.multiple_of(lo * B_, B_), B_)],
        idxv.at[pl.ds(pl.multiple_of((lo % 2) * B_, B_), B_)],
        isem.at[lo % 2])
    return lax.fori_loop(lo, hi, row, cnt0)


def _body(qT_hbm, cT_hbm, tT_hbm, qout_hbm, cout_hbm,
          tv, idxv, stg, isem, ssem):
    wid = lax.axis_index("s") * 2 + lax.axis_index("c")

    pltpu.sync_copy(tT_hbm, tv)

    drain = qout_hbm.at[0, :, pl.ds(0, 128)]
    cnt = _run_rows(qT_hbm, qout_hbm, tv, idxv, stg, isem, ssem,
                    (wid * QR) // NW, ((wid + 1) * QR) // NW, 0, drain)
    cnt = _run_rows(cT_hbm, cout_hbm, tv, idxv, stg, isem, ssem,
                    (wid * CR) // NW, ((wid + 1) * CR) // NW, cnt, drain)

    for s in range(NS):
        pltpu.make_async_copy(stg.at[s], drain, ssem.at[s]).wait()


@jax.jit
def _run(qT, cT, tT):
    mesh = plsc.VectorSubcoreMesh(core_axis_name="c", subcore_axis_name="s")
    f = pl.kernel(
        _body,
        out_type=(
            jax.ShapeDtypeStruct((QR, EMB_, B_), jnp.float32),
            jax.ShapeDtypeStruct((CR, EMB_, B_), jnp.float32),
        ),
        mesh=mesh,
        scratch_types=[
            pltpu.VMEM((TV,), jnp.float32),
            pltpu.VMEM((2 * B_,), jnp.int32),
            pltpu.VMEM((NS, EMB_, 128), jnp.float32),
            pltpu.SemaphoreType.DMA((2,)),
            pltpu.SemaphoreType.DMA((NS,)),
        ],
        compiler_params=pltpu.CompilerParams(use_tc_tiling_on_sc=True,
                                             needs_layout_passes=False),
    )
    return f(qT, cT, tT)


def kernel(question, context, char_table):
    qT = jnp.transpose(question, (1, 2, 0)).reshape(-1).astype(jnp.int32)
    cT = jnp.transpose(context, (1, 2, 0)).reshape(-1).astype(jnp.int32)
    tT = jnp.pad(char_table.T, ((0, 0), (0, B_ - VOCAB_))).reshape(-1)
    qoT, coT = _run(qT, cT, tT)
    q_emb = qoT.reshape(20, 20, EMB_, B_).transpose(3, 0, 1, 2)
    c_emb = coT.reshape(50, 20, EMB_, B_).transpose(3, 0, 1, 2)
    return (q_emb, c_emb)
